# baseline (speedup 1.0000x reference)
import functools

import jax
import jax.numpy as jnp
from jax import lax
from jax.experimental import pallas as pl
from jax.experimental.pallas import tpu as pltpu
from jax.experimental.pallas import tpu_sc as plsc

B = 4
S = 8192
D = 1024

_NC = 2
_NS = 16
_NW = _NC * _NS
_LANES = 16

_SEQ_PER_W = S // _NW
_R = 32
_TILES = _SEQ_PER_W // _R
_TILE_WORDS = _R * D
_CHUNKS = _TILE_WORDS // _LANES
_UNROLL = 8


def _sc_body(x_hbm, w_hbm, o_hbm, xv, wv):
    c = lax.axis_index("c")
    s = lax.axis_index("s")
    wid = s * _NC + c
    seq0 = wid * _SEQ_PER_W

    def tile_loop(t, carry):
        woff = (seq0 + t * _R) * D
        pltpu.sync_copy(w_hbm.at[pl.ds(woff, _TILE_WORDS)], wv)

        def batch_loop(b, carry2):
            xoff = b * (S * D) + woff
            pltpu.sync_copy(x_hbm.at[pl.ds(xoff, _TILE_WORDS)], xv)

            def add_loop(i, carry3):
                base = i * (_LANES * _UNROLL)
                for u in range(_UNROLL):
                    sl = pl.ds(base + u * _LANES, _LANES)
                    xv[sl] = xv[sl] + wv[sl]
                return carry3

            lax.fori_loop(0, _CHUNKS // _UNROLL, add_loop, 0)
            pltpu.sync_copy(xv, o_hbm.at[pl.ds(xoff, _TILE_WORDS)])
            return carry2

        lax.fori_loop(0, B, batch_loop, 0)
        return carry

    lax.fori_loop(0, _TILES, tile_loop, 0)


def kernel(x, W):
    mesh = plsc.VectorSubcoreMesh(core_axis_name="c", subcore_axis_name="s")
    run = functools.partial(
        pl.kernel,
        out_type=jax.ShapeDtypeStruct((B * S * D,), jnp.float32),
        mesh=mesh,
        scratch_types=[
            pltpu.VMEM((_TILE_WORDS,), jnp.float32),
            pltpu.VMEM((_TILE_WORDS,), jnp.float32),
        ],
    )(_sc_body)
    out = run(x.reshape(B * S * D), W.reshape(S * D))
    return out.reshape(B, S, D)


# speedup vs baseline: 1.8094x; 1.8094x over previous
import functools

import jax
import jax.numpy as jnp
from jax import lax
from jax.experimental import pallas as pl
from jax.experimental.pallas import tpu as pltpu
from jax.experimental.pallas import tpu_sc as plsc

B = 4
S = 8192
D = 1024

_BS = 512
_S_SC = 1024
_S_TC = S - _S_SC

_NC = 2
_NS = 16
_NW = _NC * _NS
_LANES = 16

_SEQ_PER_W = _S_SC // _NW
_R = 32
_TILES = _SEQ_PER_W // _R
_TILE_WORDS = _R * D
_CHUNKS = _TILE_WORDS // _LANES
_UNROLL = 8


def _tc_add(x_ref, w_ref, o_ref):
    o_ref[...] = x_ref[...] + w_ref[...]


def _sc_body(x_hbm, w_hbm, o_hbm, xv, wv):
    c = lax.axis_index("c")
    s = lax.axis_index("s")
    wid = s * _NC + c
    seq0 = _S_TC + wid * _SEQ_PER_W

    def tile_loop(t, carry):
        woff = (seq0 + t * _R) * D
        pltpu.sync_copy(w_hbm.at[pl.ds(woff, _TILE_WORDS)], wv)

        def batch_loop(b, carry2):
            xoff = b * (S * D) + woff
            ooff = b * (_S_SC * D) + (woff - _S_TC * D)
            pltpu.sync_copy(x_hbm.at[pl.ds(xoff, _TILE_WORDS)], xv)

            def add_loop(i, carry3):
                base = i * (_LANES * _UNROLL)
                for u in range(_UNROLL):
                    sl = pl.ds(base + u * _LANES, _LANES)
                    xv[sl] = xv[sl] + wv[sl]
                return carry3

            lax.fori_loop(0, _CHUNKS // _UNROLL, add_loop, 0)
            pltpu.sync_copy(xv, o_hbm.at[pl.ds(ooff, _TILE_WORDS)])
            return carry2

        lax.fori_loop(0, B, batch_loop, 0)
        return carry

    lax.fori_loop(0, _TILES, tile_loop, 0)


def kernel(x, W):
    mesh = plsc.VectorSubcoreMesh(core_axis_name="c", subcore_axis_name="s")
    sc_run = functools.partial(
        pl.kernel,
        out_type=jax.ShapeDtypeStruct((B * _S_SC * D,), jnp.float32),
        mesh=mesh,
        scratch_types=[
            pltpu.VMEM((_TILE_WORDS,), jnp.float32),
            pltpu.VMEM((_TILE_WORDS,), jnp.float32),
        ],
    )(_sc_body)
    sc_out = sc_run(x.reshape(B * S * D), W.reshape(S * D))

    tc_out = pl.pallas_call(
        _tc_add,
        grid=(_S_TC // _BS,),
        in_specs=[
            pl.BlockSpec((B, _BS, D), lambda i: (0, i, 0)),
            pl.BlockSpec((_BS, D), lambda i: (i, 0)),
        ],
        out_specs=pl.BlockSpec((B, _BS, D), lambda i: (0, i, 0)),
        out_shape=jax.ShapeDtypeStruct((B, S, D), x.dtype),
    )(x, W)

    return lax.dynamic_update_slice(
        tc_out, sc_out.reshape(B, _S_SC, D), (0, _S_TC, 0)
    )


# speedup vs baseline: 5.1198x; 2.8295x over previous
import jax
import jax.numpy as jnp
from jax.experimental import pallas as pl


_BS = 256


def _add_kernel(x_ref, w_ref, o_ref):
    o_ref[...] = x_ref[...] + w_ref[...]


def kernel(x, W):
    B, S, D = x.shape
    grid = (S // _BS,)
    return pl.pallas_call(
        _add_kernel,
        grid=grid,
        in_specs=[
            pl.BlockSpec((B, _BS, D), lambda i: (0, i, 0)),
            pl.BlockSpec((_BS, D), lambda i: (i, 0)),
        ],
        out_specs=pl.BlockSpec((B, _BS, D), lambda i: (0, i, 0)),
        out_shape=jax.ShapeDtypeStruct((B, S, D), x.dtype),
    )(x, W)
